# Initial kernel scaffold; baseline (speedup 1.0000x reference)
import jax
import jax.numpy as jnp
from jax.experimental import pallas as pl


def kernel(tree_data, W_top, W_left, W_right, bias, pair_center, pair_node, twc, lwc, rwc, hov_count):
    raise NotImplementedError("write your pallas kernel here")


# speedup vs baseline: 3.3588x; 3.3588x over previous
import functools

import jax
import jax.numpy as jnp
from jax import lax
from jax.experimental import pallas as pl
from jax.experimental.pallas import tpu as pltpu
from jax.experimental.pallas import tpu_sc as plsc

N_NODES = 100000
F = 128
NUM_TILES = 16
EPT = N_NODES // NUM_TILES
SCAT = 125
NBATCH = EPT // SCAT
CH = 625
NCHUNK = EPT // CH
BPC = CH // SCAT
NSLICE = F // 16
ACC_ROWS = N_NODES + 8


def _sc_segment_sums(x, parent3d, w_edge, zeros_tile):
  mesh = plsc.VectorSubcoreMesh(core_axis_name="c", subcore_axis_name="s")

  @functools.partial(
      pl.kernel,
      out_type=[
          jax.ShapeDtypeStruct((N_NODES, F), jnp.float32),
          jax.ShapeDtypeStruct((N_NODES, F), jnp.float32),
      ],
      mesh=mesh,
      compiler_params=pltpu.CompilerParams(use_tc_tiling_on_sc=False,
                                           needs_layout_passes=False),
      scratch_types=[
          pltpu.VMEM_SHARED((ACC_ROWS, 16), jnp.float32),
          pltpu.VMEM((CH, 16), jnp.float32),
          pltpu.VMEM((NBATCH, SCAT), jnp.int32),
          pltpu.VMEM((EPT,), jnp.float32),
      ],
  )
  def k(x_hbm, par_hbm, w_hbm, z_hbm, p_out, pr_out, acc, xbuf, idxbuf, wbuf):
    c = lax.axis_index("c")
    s = lax.axis_index("s")
    lo = s * EPT

    pltpu.sync_copy(par_hbm.at[s], idxbuf)

    @pl.when(c == 1)
    def _():
      pltpu.sync_copy(w_hbm.at[s], wbuf)

    for fs in range(NSLICE):
      pltpu.sync_copy(z_hbm, acc.at[pl.ds(lo, EPT)])
      plsc.subcore_barrier()

      @pl.loop(0, NCHUNK)
      def _(k):
        pltpu.sync_copy(
            x_hbm.at[pl.ds(lo + k * CH, CH), pl.ds(fs * 16, 16)], xbuf)

        @pl.when(c == 1)
        def _():
          @pl.loop(0, CH, unroll=5)
          def _(i):
            wl = plsc.load_gather(
                wbuf, [jnp.full((16,), k * CH + i, jnp.int32)])
            xbuf[i, :] = xbuf[i, :] * wl

        @pl.loop(0, BPC)
        def _(j):
          pltpu.sync_copy(xbuf.at[pl.ds(j * SCAT, SCAT)],
                          acc.at[idxbuf.at[k * BPC + j]], add=True)

      plsc.subcore_barrier()

      @pl.when(c == 0)
      def _():
        pltpu.sync_copy(acc.at[pl.ds(lo, EPT)],
                        p_out.at[pl.ds(lo, EPT), pl.ds(fs * 16, 16)])

      @pl.when(c == 1)
      def _():
        pltpu.sync_copy(acc.at[pl.ds(lo, EPT)],
                        pr_out.at[pl.ds(lo, EPT), pl.ds(fs * 16, 16)])

      plsc.subcore_barrier()

  return k(x, parent3d, w_edge, zeros_tile)


def _tc_body(x_ref, p_ref, pr_ref, wt_ref, wl_ref, wr_ref, b_ref,
             t_ref, r0_ref, h_ref, o_ref):
  xb = x_ref[...]
  t = t_ref[0, 0, :][:, None]
  r0 = r0_ref[0, 0, :][:, None]
  h = h_ref[0, 0, :][:, None]
  s = xb + p_ref[...]
  r = r0 * xb + pr_ref[...]
  dn = (((1,), (1,)), ((), ()))
  acc = lax.dot_general(t * s, wt_ref[...], dn, precision=lax.Precision.HIGHEST,
                        preferred_element_type=jnp.float32)
  acc += lax.dot_general((1.0 - t) * (s - r), wl_ref[...], dn,
                         precision=lax.Precision.HIGHEST,
                         preferred_element_type=jnp.float32)
  acc += lax.dot_general(r, wr_ref[...], dn, precision=lax.Precision.HIGHEST,
                         preferred_element_type=jnp.float32)
  o_ref[...] = jnp.tanh(acc + h * b_ref[0, :][None, :])


def _tc_combine(x, p, pr, wt, wl, wr, bias, twc0, rwc0, hov):
  blk = 800
  grid = N_NODES // blk
  row_spec = pl.BlockSpec((blk, F), lambda i: (i, 0))
  w_spec = pl.BlockSpec((F, F), lambda i: (0, 0))
  vec_spec = pl.BlockSpec((1, 1, blk), lambda i: (i, 0, 0))
  return pl.pallas_call(
      _tc_body,
      grid=(grid,),
      in_specs=[row_spec, row_spec, row_spec, w_spec, w_spec, w_spec,
                pl.BlockSpec((1, F), lambda i: (0, 0)),
                vec_spec, vec_spec, vec_spec],
      out_specs=row_spec,
      out_shape=jax.ShapeDtypeStruct((N_NODES, F), jnp.float32),
  )(x, p, pr, wt, wl, wr, bias.reshape(1, F),
    twc0.reshape(grid, 1, blk), rwc0.reshape(grid, 1, blk),
    hov.reshape(grid, 1, blk))


def kernel(tree_data, W_top, W_left, W_right, bias, pair_center, pair_node,
           twc, lwc, rwc, hov_count):
  n = N_NODES
  parent = jnp.concatenate(
      [jnp.full((1,), n, jnp.int32), pair_center[n:]]).reshape(
          NUM_TILES, NBATCH, SCAT)
  w_edge = jnp.concatenate(
      [jnp.zeros((1,), jnp.float32), rwc[n:]]).reshape(NUM_TILES, EPT)
  zeros_tile = jnp.zeros((EPT, 16), jnp.float32)

  p_sum, pr_sum = _sc_segment_sums(tree_data, parent, w_edge, zeros_tile)
  return _tc_combine(tree_data, p_sum, pr_sum, W_top, W_left, W_right, bias,
                     twc[:n], rwc[:n], hov_count)


# speedup vs baseline: 3.8118x; 1.1349x over previous
import functools

import jax
import jax.numpy as jnp
from jax import lax
from jax.experimental import pallas as pl
from jax.experimental.pallas import tpu as pltpu
from jax.experimental.pallas import tpu_sc as plsc

N_NODES = 100000
F = 128
NUM_TILES = 16
EPT = N_NODES // NUM_TILES
SCAT = 125
NBATCH = EPT // SCAT
CH = 625
NCHUNK = EPT // CH
BPC = CH // SCAT
NSLICE = F // 16
ACC_ROWS = N_NODES + 8


def _sc_segment_sums(x, xw, parent3d, zeros_tile):
  mesh = plsc.VectorSubcoreMesh(core_axis_name="c", subcore_axis_name="s")

  @functools.partial(
      pl.kernel,
      out_type=[
          jax.ShapeDtypeStruct((N_NODES, F), jnp.float32),
          jax.ShapeDtypeStruct((N_NODES, F), jnp.float32),
      ],
      mesh=mesh,
      compiler_params=pltpu.CompilerParams(use_tc_tiling_on_sc=False,
                                           needs_layout_passes=False),
      scratch_types=[
          pltpu.VMEM_SHARED((ACC_ROWS, 16), jnp.float32),
          pltpu.VMEM((CH, 16), jnp.float32),
          pltpu.VMEM((NBATCH, SCAT), jnp.int32),
      ],
  )
  def k(x_hbm, xw_hbm, par_hbm, z_hbm, p_out, pr_out, acc, xbuf, idxbuf):
    c = lax.axis_index("c")
    s = lax.axis_index("s")
    lo = s * EPT

    pltpu.sync_copy(par_hbm.at[s], idxbuf)

    for fs in range(NSLICE):
      pltpu.sync_copy(z_hbm, acc.at[pl.ds(lo, EPT)])
      plsc.subcore_barrier()

      @pl.loop(0, NCHUNK)
      def _(k):
        @pl.when(c == 0)
        def _():
          pltpu.sync_copy(
              x_hbm.at[pl.ds(lo + k * CH, CH), pl.ds(fs * 16, 16)], xbuf)

        @pl.when(c == 1)
        def _():
          pltpu.sync_copy(
              xw_hbm.at[pl.ds(lo + k * CH, CH), pl.ds(fs * 16, 16)], xbuf)

        @pl.loop(0, BPC)
        def _(j):
          pltpu.sync_copy(xbuf.at[pl.ds(j * SCAT, SCAT)],
                          acc.at[idxbuf.at[k * BPC + j]], add=True)

      plsc.subcore_barrier()

      @pl.when(c == 0)
      def _():
        pltpu.sync_copy(acc.at[pl.ds(lo, EPT)],
                        p_out.at[pl.ds(lo, EPT), pl.ds(fs * 16, 16)])

      @pl.when(c == 1)
      def _():
        pltpu.sync_copy(acc.at[pl.ds(lo, EPT)],
                        pr_out.at[pl.ds(lo, EPT), pl.ds(fs * 16, 16)])

      plsc.subcore_barrier()

  return k(x, xw, parent3d, zeros_tile)


def _scale_body(x_ref, w_ref, o_ref):
  o_ref[...] = x_ref[...] * w_ref[0, 0, :][:, None]


def _tc_scale(x, w_node):
  blk = 2000
  grid = N_NODES // blk
  row_spec = pl.BlockSpec((blk, F), lambda i: (i, 0))
  return pl.pallas_call(
      _scale_body,
      grid=(grid,),
      in_specs=[row_spec, pl.BlockSpec((1, 1, blk), lambda i: (i, 0, 0))],
      out_specs=row_spec,
      out_shape=jax.ShapeDtypeStruct((N_NODES, F), jnp.float32),
  )(x, w_node.reshape(grid, 1, blk))


def _tc_body(x_ref, p_ref, pr_ref, wt_ref, wl_ref, wr_ref, b_ref,
             t_ref, r0_ref, h_ref, o_ref):
  xb = x_ref[...]
  t = t_ref[0, 0, :][:, None]
  r0 = r0_ref[0, 0, :][:, None]
  h = h_ref[0, 0, :][:, None]
  s = xb + p_ref[...]
  r = r0 * xb + pr_ref[...]
  dn = (((1,), (1,)), ((), ()))
  acc = lax.dot_general(t * s, wt_ref[...], dn, precision=lax.Precision.HIGHEST,
                        preferred_element_type=jnp.float32)
  acc += lax.dot_general((1.0 - t) * (s - r), wl_ref[...], dn,
                         precision=lax.Precision.HIGHEST,
                         preferred_element_type=jnp.float32)
  acc += lax.dot_general(r, wr_ref[...], dn, precision=lax.Precision.HIGHEST,
                         preferred_element_type=jnp.float32)
  o_ref[...] = jnp.tanh(acc + h * b_ref[0, :][None, :])


def _tc_combine(x, p, pr, wt, wl, wr, bias, twc0, rwc0, hov):
  blk = 800
  grid = N_NODES // blk
  row_spec = pl.BlockSpec((blk, F), lambda i: (i, 0))
  w_spec = pl.BlockSpec((F, F), lambda i: (0, 0))
  vec_spec = pl.BlockSpec((1, 1, blk), lambda i: (i, 0, 0))
  return pl.pallas_call(
      _tc_body,
      grid=(grid,),
      in_specs=[row_spec, row_spec, row_spec, w_spec, w_spec, w_spec,
                pl.BlockSpec((1, F), lambda i: (0, 0)),
                vec_spec, vec_spec, vec_spec],
      out_specs=row_spec,
      out_shape=jax.ShapeDtypeStruct((N_NODES, F), jnp.float32),
  )(x, p, pr, wt, wl, wr, bias.reshape(1, F),
    twc0.reshape(grid, 1, blk), rwc0.reshape(grid, 1, blk),
    hov.reshape(grid, 1, blk))


def kernel(tree_data, W_top, W_left, W_right, bias, pair_center, pair_node,
           twc, lwc, rwc, hov_count):
  n = N_NODES
  parent = jnp.concatenate(
      [jnp.full((1,), n, jnp.int32), pair_center[n:]]).reshape(
          NUM_TILES, NBATCH, SCAT)
  w_node = jnp.concatenate([jnp.zeros((1,), jnp.float32), rwc[n:]])
  zeros_tile = jnp.zeros((EPT, 16), jnp.float32)

  xw = _tc_scale(tree_data, w_node)
  p_sum, pr_sum = _sc_segment_sums(tree_data, xw, parent, zeros_tile)
  return _tc_combine(tree_data, p_sum, pr_sum, W_top, W_left, W_right, bias,
                     twc[:n], rwc[:n], hov_count)


# speedup vs baseline: 3.8286x; 1.0044x over previous
import functools

import jax
import jax.numpy as jnp
from jax import lax
from jax.experimental import pallas as pl
from jax.experimental.pallas import tpu as pltpu
from jax.experimental.pallas import tpu_sc as plsc

N_NODES = 100000
F = 128
NUM_TILES = 16
EPT = N_NODES // NUM_TILES
SCAT = 125
NBATCH = EPT // SCAT
CH = 625
NCHUNK = EPT // CH
BPC = CH // SCAT
NSLICE = F // 16
ACC_ROWS = N_NODES + 8


def _sc_segment_sums(x, xw, parent3d):
  mesh = plsc.VectorSubcoreMesh(core_axis_name="c", subcore_axis_name="s")

  @functools.partial(
      pl.kernel,
      out_type=[
          jax.ShapeDtypeStruct((N_NODES, F), jnp.float32),
          jax.ShapeDtypeStruct((N_NODES, F), jnp.float32),
      ],
      mesh=mesh,
      compiler_params=pltpu.CompilerParams(use_tc_tiling_on_sc=False,
                                           needs_layout_passes=False),
      scratch_types=[
          pltpu.VMEM_SHARED((ACC_ROWS, 16), jnp.float32),
          pltpu.VMEM((CH, 16), jnp.float32),
          pltpu.VMEM((NBATCH, SCAT), jnp.int32),
          pltpu.VMEM((CH, 16), jnp.float32),
      ],
  )
  def k(x_hbm, xw_hbm, par_hbm, p_out, pr_out, acc, xbuf, idxbuf, zbuf):
    c = lax.axis_index("c")
    s = lax.axis_index("s")
    lo = s * EPT

    pltpu.sync_copy(par_hbm.at[s], idxbuf)

    @pl.loop(0, CH)
    def _(i):
      zbuf[i, :] = jnp.zeros((16,), jnp.float32)

    @pl.loop(0, NCHUNK)
    def _(k):
      pltpu.sync_copy(zbuf, acc.at[pl.ds(lo + k * CH, CH)])

    for fs in range(NSLICE):
      plsc.subcore_barrier()

      @pl.loop(0, NCHUNK)
      def _(k):
        @pl.when(c == 0)
        def _():
          pltpu.sync_copy(
              x_hbm.at[pl.ds(lo + k * CH, CH), pl.ds(fs * 16, 16)], xbuf)

        @pl.when(c == 1)
        def _():
          pltpu.sync_copy(
              xw_hbm.at[pl.ds(lo + k * CH, CH), pl.ds(fs * 16, 16)], xbuf)

        @pl.loop(0, BPC)
        def _(j):
          pltpu.sync_copy(xbuf.at[pl.ds(j * SCAT, SCAT)],
                          acc.at[idxbuf.at[k * BPC + j]], add=True)

      plsc.subcore_barrier()

      @pl.when(c == 0)
      def _():
        pltpu.sync_copy(acc.at[pl.ds(lo, EPT)],
                        p_out.at[pl.ds(lo, EPT), pl.ds(fs * 16, 16)])

      @pl.when(c == 1)
      def _():
        pltpu.sync_copy(acc.at[pl.ds(lo, EPT)],
                        pr_out.at[pl.ds(lo, EPT), pl.ds(fs * 16, 16)])

      if fs != NSLICE - 1:
        @pl.loop(0, NCHUNK)
        def _(k):
          pltpu.sync_copy(zbuf, acc.at[pl.ds(lo + k * CH, CH)])

  return k(x, xw, parent3d)


def _scale_body(x_ref, w_ref, o_ref):
  o_ref[...] = x_ref[...] * w_ref[0, 0, :][:, None]


def _tc_scale(x, w_node):
  blk = 2000
  grid = N_NODES // blk
  row_spec = pl.BlockSpec((blk, F), lambda i: (i, 0))
  return pl.pallas_call(
      _scale_body,
      grid=(grid,),
      in_specs=[row_spec, pl.BlockSpec((1, 1, blk), lambda i: (i, 0, 0))],
      out_specs=row_spec,
      out_shape=jax.ShapeDtypeStruct((N_NODES, F), jnp.float32),
  )(x, w_node.reshape(grid, 1, blk))


def _tc_body(x_ref, p_ref, pr_ref, wt_ref, wl_ref, wr_ref, b_ref,
             t_ref, r0_ref, h_ref, o_ref):
  xb = x_ref[...]
  t = t_ref[0, 0, :][:, None]
  r0 = r0_ref[0, 0, :][:, None]
  h = h_ref[0, 0, :][:, None]
  s = xb + p_ref[...]
  r = r0 * xb + pr_ref[...]
  dn = (((1,), (1,)), ((), ()))
  acc = lax.dot_general(t * s, wt_ref[...], dn, precision=lax.Precision.HIGHEST,
                        preferred_element_type=jnp.float32)
  acc += lax.dot_general((1.0 - t) * (s - r), wl_ref[...], dn,
                         precision=lax.Precision.HIGHEST,
                         preferred_element_type=jnp.float32)
  acc += lax.dot_general(r, wr_ref[...], dn, precision=lax.Precision.HIGHEST,
                         preferred_element_type=jnp.float32)
  o_ref[...] = jnp.tanh(acc + h * b_ref[0, :][None, :])


def _tc_combine(x, p, pr, wt, wl, wr, bias, twc0, rwc0, hov):
  blk = 800
  grid = N_NODES // blk
  row_spec = pl.BlockSpec((blk, F), lambda i: (i, 0))
  w_spec = pl.BlockSpec((F, F), lambda i: (0, 0))
  vec_spec = pl.BlockSpec((1, 1, blk), lambda i: (i, 0, 0))
  return pl.pallas_call(
      _tc_body,
      grid=(grid,),
      in_specs=[row_spec, row_spec, row_spec, w_spec, w_spec, w_spec,
                pl.BlockSpec((1, F), lambda i: (0, 0)),
                vec_spec, vec_spec, vec_spec],
      out_specs=row_spec,
      out_shape=jax.ShapeDtypeStruct((N_NODES, F), jnp.float32),
  )(x, p, pr, wt, wl, wr, bias.reshape(1, F),
    twc0.reshape(grid, 1, blk), rwc0.reshape(grid, 1, blk),
    hov.reshape(grid, 1, blk))


def kernel(tree_data, W_top, W_left, W_right, bias, pair_center, pair_node,
           twc, lwc, rwc, hov_count):
  n = N_NODES
  parent = jnp.concatenate(
      [jnp.full((1,), n, jnp.int32), pair_center[n:]]).reshape(
          NUM_TILES, NBATCH, SCAT)
  w_node = jnp.concatenate([jnp.zeros((1,), jnp.float32), rwc[n:]])

  xw = _tc_scale(tree_data, w_node)
  p_sum, pr_sum = _sc_segment_sums(tree_data, xw, parent)
  return _tc_combine(tree_data, p_sum, pr_sum, W_top, W_left, W_right, bias,
                     twc[:n], rwc[:n], hov_count)


# speedup vs baseline: 4.0542x; 1.0589x over previous
import functools

import jax
import jax.numpy as jnp
from jax import lax
from jax.experimental import pallas as pl
from jax.experimental.pallas import tpu as pltpu
from jax.experimental.pallas import tpu_sc as plsc

N_NODES = 100000
F = 128
NUM_TILES = 16
EPT = N_NODES // NUM_TILES
SCAT = 125
NBATCH = EPT // SCAT
CH = 625
NCHUNK = EPT // CH
BPC = CH // SCAT
NSLICE = F // 16
ACC_ROWS = N_NODES + 8


def _sc_segment_sums(x, xw, parent3d):
  mesh = plsc.VectorSubcoreMesh(core_axis_name="c", subcore_axis_name="s")

  @functools.partial(
      pl.kernel,
      out_type=[
          jax.ShapeDtypeStruct((N_NODES, F), jnp.float32),
          jax.ShapeDtypeStruct((N_NODES, F), jnp.float32),
      ],
      mesh=mesh,
      compiler_params=pltpu.CompilerParams(use_tc_tiling_on_sc=False,
                                           needs_layout_passes=False),
      scratch_types=[
          pltpu.VMEM_SHARED((ACC_ROWS, 16), jnp.float32),
          pltpu.VMEM((CH, 16), jnp.float32),
          pltpu.VMEM((CH, 16), jnp.float32),
          pltpu.VMEM((NBATCH, SCAT), jnp.int32),
          pltpu.SemaphoreType.DMA,
          pltpu.SemaphoreType.DMA,
          pltpu.SemaphoreType.DMA,
      ],
  )
  def k(x_hbm, xw_hbm, par_hbm, p_out, pr_out, acc, buf0, buf1, idxbuf,
        sem_a, sem_b, sem_s):
    c = lax.axis_index("c")
    s = lax.axis_index("s")
    lo = s * EPT

    pltpu.sync_copy(par_hbm.at[s], idxbuf)

    def zero_my_slab():
      @pl.loop(0, CH)
      def _(i):
        buf0[i, :] = jnp.zeros((16,), jnp.float32)
      zs = [pltpu.async_copy(buf0, acc.at[pl.ds(lo + q * CH, CH)], sem_s)
            for q in range(NCHUNK)]
      for z in zs:
        z.wait()

    zero_my_slab()

    for fs in range(NSLICE):
      plsc.subcore_barrier()

      src = [x_hbm, xw_hbm]
      col = pl.ds(fs * 16, 16)

      @pl.loop(0, NCHUNK // 2)
      def _(p):
        a = 2 * p
        b = 2 * p + 1
        for cc in range(2):
          @pl.when(c == cc)
          def _():
            pltpu.async_copy(
                src[cc].at[pl.ds(lo + a * CH, CH), col], buf0, sem_a)
            pltpu.async_copy(
                src[cc].at[pl.ds(lo + b * CH, CH), col], buf1, sem_b)

        pltpu.make_async_copy(
            x_hbm.at[pl.ds(lo + a * CH, CH), col], buf0, sem_a).wait()
        sc = [pltpu.async_copy(buf0.at[pl.ds(j * SCAT, SCAT)],
                               acc.at[idxbuf.at[a * BPC + j]], sem_s,
                               add=True)
              for j in range(BPC)]
        pltpu.make_async_copy(
            x_hbm.at[pl.ds(lo + b * CH, CH), col], buf1, sem_b).wait()
        sc += [pltpu.async_copy(buf1.at[pl.ds(j * SCAT, SCAT)],
                                acc.at[idxbuf.at[b * BPC + j]], sem_s,
                                add=True)
               for j in range(BPC)]
        for d in sc:
          d.wait()

      plsc.subcore_barrier()

      @pl.when(c == 0)
      def _():
        pltpu.sync_copy(acc.at[pl.ds(lo, EPT)],
                        p_out.at[pl.ds(lo, EPT), pl.ds(fs * 16, 16)])

      @pl.when(c == 1)
      def _():
        pltpu.sync_copy(acc.at[pl.ds(lo, EPT)],
                        pr_out.at[pl.ds(lo, EPT), pl.ds(fs * 16, 16)])

      if fs != NSLICE - 1:
        zero_my_slab()

  return k(x, xw, parent3d)


def _scale_body(x_ref, w_ref, o_ref):
  o_ref[...] = x_ref[...] * w_ref[0, 0, :][:, None]


def _tc_scale(x, w_node):
  blk = 2000
  grid = N_NODES // blk
  row_spec = pl.BlockSpec((blk, F), lambda i: (i, 0))
  return pl.pallas_call(
      _scale_body,
      grid=(grid,),
      in_specs=[row_spec, pl.BlockSpec((1, 1, blk), lambda i: (i, 0, 0))],
      out_specs=row_spec,
      out_shape=jax.ShapeDtypeStruct((N_NODES, F), jnp.float32),
  )(x, w_node.reshape(grid, 1, blk))


def _tc_body(x_ref, p_ref, pr_ref, wt_ref, wl_ref, wr_ref, b_ref,
             t_ref, r0_ref, h_ref, o_ref):
  xb = x_ref[...]
  t = t_ref[0, 0, :][:, None]
  r0 = r0_ref[0, 0, :][:, None]
  h = h_ref[0, 0, :][:, None]
  s = xb + p_ref[...]
  r = r0 * xb + pr_ref[...]
  dn = (((1,), (1,)), ((), ()))
  acc = lax.dot_general(t * s, wt_ref[...], dn, precision=lax.Precision.HIGHEST,
                        preferred_element_type=jnp.float32)
  acc += lax.dot_general((1.0 - t) * (s - r), wl_ref[...], dn,
                         precision=lax.Precision.HIGHEST,
                         preferred_element_type=jnp.float32)
  acc += lax.dot_general(r, wr_ref[...], dn, precision=lax.Precision.HIGHEST,
                         preferred_element_type=jnp.float32)
  o_ref[...] = jnp.tanh(acc + h * b_ref[0, :][None, :])


def _tc_combine(x, p, pr, wt, wl, wr, bias, twc0, rwc0, hov):
  blk = 800
  grid = N_NODES // blk
  row_spec = pl.BlockSpec((blk, F), lambda i: (i, 0))
  w_spec = pl.BlockSpec((F, F), lambda i: (0, 0))
  vec_spec = pl.BlockSpec((1, 1, blk), lambda i: (i, 0, 0))
  return pl.pallas_call(
      _tc_body,
      grid=(grid,),
      in_specs=[row_spec, row_spec, row_spec, w_spec, w_spec, w_spec,
                pl.BlockSpec((1, F), lambda i: (0, 0)),
                vec_spec, vec_spec, vec_spec],
      out_specs=row_spec,
      out_shape=jax.ShapeDtypeStruct((N_NODES, F), jnp.float32),
  )(x, p, pr, wt, wl, wr, bias.reshape(1, F),
    twc0.reshape(grid, 1, blk), rwc0.reshape(grid, 1, blk),
    hov.reshape(grid, 1, blk))


def kernel(tree_data, W_top, W_left, W_right, bias, pair_center, pair_node,
           twc, lwc, rwc, hov_count):
  n = N_NODES
  parent = jnp.concatenate(
      [jnp.full((1,), n, jnp.int32), pair_center[n:]]).reshape(
          NUM_TILES, NBATCH, SCAT)
  w_node = jnp.concatenate([jnp.zeros((1,), jnp.float32), rwc[n:]])

  xw = _tc_scale(tree_data, w_node)
  p_sum, pr_sum = _sc_segment_sums(tree_data, xw, parent)
  return _tc_combine(tree_data, p_sum, pr_sum, W_top, W_left, W_right, bias,
                     twc[:n], rwc[:n], hov_count)


# speedup vs baseline: 4.3693x; 1.0777x over previous
import functools

import jax
import jax.numpy as jnp
from jax import lax
from jax.experimental import pallas as pl
from jax.experimental.pallas import tpu as pltpu
from jax.experimental.pallas import tpu_sc as plsc

N_NODES = 100000
F = 128
NUM_TILES = 16
EPT = N_NODES // NUM_TILES
SCAT = 125
NBATCH = EPT // SCAT
CH = 625
NCHUNK = EPT // CH
BPC = CH // SCAT
NSLICE = F // 16
ACC_ROWS = N_NODES + 8


def _sc_segment_sums(x, xw, parent3d):
  mesh = plsc.VectorSubcoreMesh(core_axis_name="c", subcore_axis_name="s")

  @functools.partial(
      pl.kernel,
      out_type=[
          jax.ShapeDtypeStruct((N_NODES, F), jnp.float32),
          jax.ShapeDtypeStruct((N_NODES, F), jnp.float32),
      ],
      mesh=mesh,
      compiler_params=pltpu.CompilerParams(use_tc_tiling_on_sc=False,
                                           needs_layout_passes=False),
      scratch_types=[
          pltpu.VMEM_SHARED((ACC_ROWS, 16), jnp.float32),
          pltpu.VMEM((CH, 16), jnp.float32),
          pltpu.VMEM((CH, 16), jnp.float32),
          pltpu.VMEM((NBATCH, SCAT), jnp.int32),
          pltpu.SemaphoreType.DMA,
          pltpu.SemaphoreType.DMA,
          pltpu.SemaphoreType.DMA,
      ],
  )
  def k(x_hbm, xw_hbm, par_hbm, p_out, pr_out, acc, buf0, buf1, idxbuf,
        sem_a, sem_b, sem_s):
    c = lax.axis_index("c")
    s = lax.axis_index("s")
    lo = s * EPT

    pltpu.sync_copy(par_hbm.at[s], idxbuf)

    def zero_my_slab():
      @pl.loop(0, CH)
      def _(i):
        buf0[i, :] = jnp.zeros((16,), jnp.float32)
      zs = [pltpu.async_copy(buf0, acc.at[pl.ds(lo + q * CH, CH)], sem_s)
            for q in range(NCHUNK)]
      for z in zs:
        z.wait()

    zero_my_slab()

    for fs in range(NSLICE):
      plsc.subcore_barrier()

      src = [x_hbm, xw_hbm]
      col = pl.ds(fs * 16, 16)

      @pl.loop(0, NCHUNK // 2)
      def _(p):
        a = 2 * p
        b = 2 * p + 1
        for cc in range(2):
          @pl.when(c == cc)
          def _():
            pltpu.async_copy(
                src[cc].at[pl.ds(lo + a * CH, CH), col], buf0, sem_a)
            pltpu.async_copy(
                src[cc].at[pl.ds(lo + b * CH, CH), col], buf1, sem_b)

        pltpu.make_async_copy(
            x_hbm.at[pl.ds(lo + a * CH, CH), col], buf0, sem_a).wait()
        sc = [pltpu.async_copy(buf0.at[pl.ds(j * SCAT, SCAT)],
                               acc.at[idxbuf.at[a * BPC + j]], sem_s,
                               add=True)
              for j in range(BPC)]
        pltpu.make_async_copy(
            x_hbm.at[pl.ds(lo + b * CH, CH), col], buf1, sem_b).wait()
        sc += [pltpu.async_copy(buf1.at[pl.ds(j * SCAT, SCAT)],
                                acc.at[idxbuf.at[b * BPC + j]], sem_s,
                                add=True)
               for j in range(BPC)]
        for d in sc:
          d.wait()

      plsc.subcore_barrier()

      @pl.when(c == 0)
      def _():
        pltpu.sync_copy(acc.at[pl.ds(lo, EPT)],
                        p_out.at[pl.ds(lo, EPT), pl.ds(fs * 16, 16)])

      @pl.when(c == 1)
      def _():
        pltpu.sync_copy(acc.at[pl.ds(lo, EPT)],
                        pr_out.at[pl.ds(lo, EPT), pl.ds(fs * 16, 16)])

      if fs != NSLICE - 1:
        zero_my_slab()

  return k(x, xw, parent3d)


def _scale_body(x_ref, w_ref, o_ref):
  o_ref[...] = x_ref[...] * w_ref[0, 0, :][:, None]


def _tc_scale(x, w_node):
  blk = 2000
  grid = N_NODES // blk
  row_spec = pl.BlockSpec((blk, F), lambda i: (i, 0))
  return pl.pallas_call(
      _scale_body,
      grid=(grid,),
      in_specs=[row_spec, pl.BlockSpec((1, 1, blk), lambda i: (i, 0, 0))],
      out_specs=row_spec,
      out_shape=jax.ShapeDtypeStruct((N_NODES, F), jnp.float32),
  )(x, w_node.reshape(grid, 1, blk))


def _tc_body(x_ref, p_ref, pr_ref, wt_ref, wl_ref, wr_ref, b_ref,
             t_ref, r0_ref, h_ref, o_ref):
  xb = x_ref[...]
  t = t_ref[0, 0, :][:, None]
  r0 = r0_ref[0, 0, :][:, None]
  h = h_ref[0, 0, :][:, None]
  s = xb + p_ref[...]
  r = r0 * xb + pr_ref[...]
  dn = (((1,), (1,)), ((), ()))
  acc = lax.dot_general(t * s, wt_ref[...], dn, precision=lax.Precision.DEFAULT,
                        preferred_element_type=jnp.float32)
  acc += lax.dot_general((1.0 - t) * (s - r), wl_ref[...], dn,
                         precision=lax.Precision.DEFAULT,
                         preferred_element_type=jnp.float32)
  acc += lax.dot_general(r, wr_ref[...], dn, precision=lax.Precision.DEFAULT,
                         preferred_element_type=jnp.float32)
  o_ref[...] = jnp.tanh(acc + h * b_ref[0, :][None, :])


def _tc_combine(x, p, pr, wt, wl, wr, bias, twc0, rwc0, hov):
  blk = 800
  grid = N_NODES // blk
  row_spec = pl.BlockSpec((blk, F), lambda i: (i, 0))
  w_spec = pl.BlockSpec((F, F), lambda i: (0, 0))
  vec_spec = pl.BlockSpec((1, 1, blk), lambda i: (i, 0, 0))
  return pl.pallas_call(
      _tc_body,
      grid=(grid,),
      in_specs=[row_spec, row_spec, row_spec, w_spec, w_spec, w_spec,
                pl.BlockSpec((1, F), lambda i: (0, 0)),
                vec_spec, vec_spec, vec_spec],
      out_specs=row_spec,
      out_shape=jax.ShapeDtypeStruct((N_NODES, F), jnp.float32),
  )(x, p, pr, wt, wl, wr, bias.reshape(1, F),
    twc0.reshape(grid, 1, blk), rwc0.reshape(grid, 1, blk),
    hov.reshape(grid, 1, blk))


def kernel(tree_data, W_top, W_left, W_right, bias, pair_center, pair_node,
           twc, lwc, rwc, hov_count):
  n = N_NODES
  parent = jnp.concatenate(
      [jnp.full((1,), n, jnp.int32), pair_center[n:]]).reshape(
          NUM_TILES, NBATCH, SCAT)
  w_node = jnp.concatenate([jnp.zeros((1,), jnp.float32), rwc[n:]])

  xw = _tc_scale(tree_data, w_node)
  p_sum, pr_sum = _sc_segment_sums(tree_data, xw, parent)
  return _tc_combine(tree_data, p_sum, pr_sum, W_top, W_left, W_right, bias,
                     twc[:n], rwc[:n], hov_count)
